# baseline (speedup 1.0000x reference)
import jax
import jax.numpy as jnp
from jax.experimental import pallas as pl
from jax.experimental.pallas import tpu as pltpu

N = 10000
B = 20
NPG = N // B
D = 128
K = 8
C = 64
EPS = 1e-5
F32 = jnp.float32


def _conv_kernel(x_ref, sqr_ref, sqc_ref, W1_ref, b1_ref, W2_ref, b2_ref,
                 h_ref):
    xb = x_ref[0]
    sqr = sqr_ref[0]
    sqc = sqc_ref[0]
    W1 = W1_ref[...]
    b1 = b1_ref[...]
    W2 = W2_ref[...]
    b2 = b2_ref[...]
    G = jax.lax.dot_general(xb, xb, (((1,), (1,)), ((), ())),
                            preferred_element_type=F32)
    dist = (sqc - 2.0 * G) + sqr
    iota_j = jax.lax.broadcasted_iota(jnp.int32, (NPG, NPG), 1)
    d = xb.shape[1]
    mask = jnp.int32(-65536)
    hi_f = jax.lax.bitcast_convert_type(
        jax.lax.bitcast_convert_type(xb, jnp.int32) & mask, F32)
    hi = hi_f.astype(jnp.bfloat16)
    r1 = xb - hi_f
    mid_f = jax.lax.bitcast_convert_type(
        jax.lax.bitcast_convert_type(r1, jnp.int32) & mask, F32)
    mid = mid_f.astype(jnp.bfloat16)
    lo = (r1 - mid_f).astype(jnp.bfloat16)
    xpack = jnp.concatenate([hi, mid, lo], axis=1)
    acc = None
    for _ in range(K):
        m = jnp.min(dist, axis=1, keepdims=True)
        sel = jnp.min(jnp.where(dist == m, iota_j, NPG), axis=1, keepdims=True)
        onehot = iota_j == sel
        y = jnp.dot(onehot.astype(jnp.bfloat16), xpack,
                    preferred_element_type=F32)
        xj = (y[:, :d] + y[:, d:2 * d]) + y[:, 2 * d:]
        e = jnp.concatenate([xb, xj - xb], axis=1)
        t = jax.nn.relu(jnp.dot(e, W1, preferred_element_type=F32) + b1)
        hk = jnp.dot(t, W2, preferred_element_type=F32) + b2
        acc = hk if acc is None else acc + hk
        dist = jnp.where(onehot, jnp.float32(jnp.inf), dist)
    h_ref[0] = jax.nn.relu(acc)


def _tail_kernel(hn_ref,
                 W0_ref, bh0_ref, g0_ref, bb0_ref,
                 W1_ref, bh1_ref, g1_ref, bb1_ref,
                 Wl_ref, bl_ref, out_ref):
    hn = hn_ref[...]
    gap = jnp.mean(hn, axis=1)
    gmp = jnp.max(hn, axis=1)
    out = jnp.concatenate([gap, gmp], axis=1)

    def bn(x, g, b):
        m = jnp.mean(x, axis=0, keepdims=True)
        v = jnp.mean((x - m) ** 2, axis=0, keepdims=True)
        return g * (x - m) / jnp.sqrt(v + EPS) + b

    out = jnp.dot(out, W0_ref[...], preferred_element_type=F32) + bh0_ref[...]
    out = bn(jax.nn.relu(out), g0_ref[...], bb0_ref[...])
    out = jnp.dot(out, W1_ref[...], preferred_element_type=F32) + bh1_ref[...]
    out = bn(jax.nn.relu(out), g1_ref[...], bb1_ref[...])
    out = jnp.dot(out, Wl_ref[...], preferred_element_type=F32) + bl_ref[...]
    y = out - jnp.max(out, axis=1, keepdims=True)
    out_ref[...] = y - jnp.log(jnp.sum(jnp.exp(y), axis=1, keepdims=True))


def _full(shape):
    return pl.BlockSpec(shape, lambda g: (0,) * len(shape))


def _rowvec(p):
    return p.reshape(1, -1)


def _conv_layer(xg, W1, b1, W2, b2, interpret=False):
    d = xg.shape[-1]
    sq = jnp.sum(xg * xg, axis=-1)
    sqr = sq.reshape(B, 1, NPG)
    sqc = sq.reshape(B, NPG, 1)
    return pl.pallas_call(
        _conv_kernel,
        grid=(B,),
        in_specs=[pl.BlockSpec((1, NPG, d), lambda g: (g, 0, 0)),
                  pl.BlockSpec((1, 1, NPG), lambda g: (g, 0, 0)),
                  pl.BlockSpec((1, NPG, 1), lambda g: (g, 0, 0)),
                  _full((2 * d, C)), _full((1, C)),
                  _full((C, C)), _full((1, C))],
        out_specs=pl.BlockSpec((1, NPG, C), lambda g: (g, 0, 0)),
        out_shape=jax.ShapeDtypeStruct((B, NPG, C), F32),
        compiler_params=pltpu.CompilerParams(
            dimension_semantics=("parallel",)),
        interpret=interpret,
    )(xg, sqr, sqc, W1, _rowvec(b1), W2, _rowvec(b2))


def _exact_colsum(x):
    M, C_ = x.shape
    P = 1 << (M - 1).bit_length()
    if P != M:
        x = jnp.concatenate([x, jnp.zeros((P - M, C_), x.dtype)], axis=0)
    err = jnp.zeros((1, C_), x.dtype)
    s = x
    while s.shape[0] > 1:
        half = s.shape[0] // 2
        a, b = s[:half], s[half:]
        t = a + b
        bv = t - a
        e = (a - (t - bv)) + (b - bv)
        err = err + jnp.sum(e, axis=0, keepdims=True)
        s = t
    return s + err


def _bn_ref(x, g, b):
    n = x.shape[0]
    m = _exact_colsum(x) / n
    v = _exact_colsum((x - m) ** 2) / n
    return g * (x - m) / jnp.sqrt(v + EPS) + b


def _run(x, params, interpret=False):
    xg = x.reshape(B, NPG, D)
    h0 = _conv_layer(xg, params['conv0_W1'], params['conv0_b1'],
                     params['conv0_W2'], params['conv0_b2'], interpret)
    x1 = _bn_ref(h0.reshape(N, C), params['conv0_bn_g'], params['conv0_bn_b'])
    h1 = _conv_layer(x1.reshape(B, NPG, C), params['conv1_W1'], params['conv1_b1'],
                     params['conv1_W2'], params['conv1_b2'], interpret)
    x2 = _bn_ref(h1.reshape(N, C), params['conv1_bn_g'], params['conv1_bn_b'])

    out = pl.pallas_call(
        _tail_kernel,
        out_shape=jax.ShapeDtypeStruct((B, 16), F32),
        interpret=interpret,
    )(x2.reshape(B, NPG, C),
      params['hl0_W'], _rowvec(params['hl0_b']),
      _rowvec(params['hl0_bn_g']), _rowvec(params['hl0_bn_b']),
      params['hl1_W'], _rowvec(params['hl1_b']),
      _rowvec(params['hl1_bn_g']), _rowvec(params['hl1_bn_b']),
      params['last_W'], _rowvec(params['last_b']))
    return out


def kernel(x, batch, params):
    del batch
    return _run(x, params)
